# baseline (speedup 1.0000x reference)
import functools

import jax
import jax.numpy as jnp
from jax import lax
from jax.experimental import pallas as pl
from jax.experimental.pallas import tpu as pltpu
from jax.experimental.pallas import tpu_sc as plsc

V = 100000
D = 64
DM = 512
B, N, T = 1024, 8, 64
BN = B * N
BNT = BN * T
TP = T // 2

NC, NS = 2, 16
NW = NC * NS
K = 8
BNK = BN // K
BNTK = BNT // K
PER_W = BNTK // NW
G = 512
NCH = PER_W // G
RPC = G // T
RPW = PER_W // T


def _sc_gather(ids_pad, table):
    mesh = plsc.VectorSubcoreMesh(core_axis_name="c", subcore_axis_name="s")

    @functools.partial(
        pl.kernel,
        out_type=jax.ShapeDtypeStruct((BNTK, D), jnp.float32),
        mesh=mesh,
        scratch_types=[
            pltpu.VMEM((RPW, 2 * T), jnp.int32),
            pltpu.VMEM((G,), jnp.int32),
            pltpu.VMEM((G,), jnp.int32),
            pltpu.VMEM((G, D), jnp.float32),
            pltpu.VMEM((G, D), jnp.float32),
            pltpu.SemaphoreType.DMA,
            pltpu.SemaphoreType.DMA,
        ],
        compiler_params=pltpu.CompilerParams(use_tc_tiling_on_sc=False),
    )
    def gather_kernel(ids_hbm, table_hbm, emb_hbm, idp_v, idx0, idx1,
                      rows0, rows1, sem0, sem1):
        wid = lax.axis_index("s") * NC + lax.axis_index("c")
        base = wid * PER_W
        idxs = (idx0, idx1)
        rows = (rows0, rows1)
        sems = (sem0, sem1)

        pltpu.sync_copy(ids_hbm.at[pl.ds(wid * RPW, RPW)], idp_v)

        def start_gather(c, b):
            for r in range(RPC):
                for k in range(T // 16):
                    src = idp_v[c * RPC + r, pl.ds(k * 16, 16)]
                    idxs[b][pl.ds(r * T + k * 16, 16)] = src
            pltpu.async_copy(table_hbm.at[idxs[b]], rows[b], sems[b])

        def drain_and_scatter(c, b):
            pltpu.make_async_copy(table_hbm.at[idxs[b]],
                                  rows[b], sems[b]).wait()
            pltpu.sync_copy(rows[b], emb_hbm.at[pl.ds(base + c * G, G)])

        start_gather(0, 0)

        def body(j, carry):
            c0 = 2 * j
            start_gather(c0 + 1, 1)
            drain_and_scatter(c0, 0)
            start_gather(c0 + 2, 0)
            drain_and_scatter(c0 + 1, 1)
            return carry

        lax.fori_loop(0, NCH // 2 - 1, body, 0)
        c0 = NCH - 2
        start_gather(c0 + 1, 1)
        drain_and_scatter(c0, 0)
        drain_and_scatter(c0 + 1, 1)

    return gather_kernel(ids_pad, table)


S = 512


def _tc_body(emb_ref, wam_ref, ba_ref, wp_ref, bp_ref, g_ref, b_ref, *rest):
    out_ref = rest[-1]
    e2 = emb_ref[...]
    lfull = jnp.dot(e2, wam_ref[...],
                    preferred_element_type=jnp.float32)
    lfull = jnp.clip(lfull + ba_ref[0, 0], -20.0, 20.0)
    ef = jnp.exp(lfull)
    e3 = e2.reshape(S, TP, 2 * D)
    ef3 = ef.reshape(S, TP, 2 * D)
    pooled_un = jnp.sum(e3 * ef3, axis=1)
    sef = jnp.sum(ef3, axis=1)
    z = sef[:, 0:1] + sef[:, D:D + 1]
    pooled = pooled_un / z
    out = jnp.dot(pooled, wp_ref[...],
                  preferred_element_type=jnp.float32) + bp_ref[...]
    mu = jnp.mean(out, axis=1, keepdims=True)
    var = jnp.mean((out - mu) ** 2, axis=1, keepdims=True)
    y = (out - mu) * lax.rsqrt(var + 1e-5)
    out_ref[...] = y * g_ref[...] + b_ref[...]


def _tc_pool_proj_ln(emb2, WaM, ba, Wp2, bp, gamma, beta, acc, koff):
    grid = (BNK // S,)
    in_specs = [
        pl.BlockSpec((S * TP, 2 * D), lambda i: (i, 0)),
        pl.BlockSpec((2 * D, 2 * D), lambda i: (0, 0)),
        pl.BlockSpec((1, 1), lambda i: (0, 0)),
        pl.BlockSpec((2 * D, DM), lambda i: (0, 0)),
        pl.BlockSpec((1, DM), lambda i: (0, 0)),
        pl.BlockSpec((1, DM), lambda i: (0, 0)),
        pl.BlockSpec((1, DM), lambda i: (0, 0)),
    ]
    args = [emb2, WaM, ba.reshape(1, 1), Wp2, bp.reshape(1, DM),
            gamma.reshape(1, DM), beta.reshape(1, DM)]
    aliases = {}
    if acc is not None:
        in_specs.append(pl.BlockSpec(memory_space=pl.ANY))
        args.append(acc)
        aliases = {7: 0}
    return pl.pallas_call(
        _tc_body,
        grid=grid,
        in_specs=in_specs,
        out_specs=pl.BlockSpec((S, DM), lambda i: (i + koff, 0)),
        out_shape=jax.ShapeDtypeStruct((BN, DM), jnp.float32),
        input_output_aliases=aliases,
    )(*args)


def kernel(ids, table, Wa, ba, Wp, bp, gamma, beta):
    ids_pad = jnp.pad(ids.astype(jnp.int32).reshape(BN, T),
                      ((0, 0), (0, T)))
    WaM = jnp.kron(jnp.eye(2, dtype=jnp.float32),
                   jnp.tile(Wa, (1, D)))
    Wp2 = jnp.concatenate([Wp, Wp], axis=0)
    out = None
    for k in range(K):
        emb = _sc_gather(ids_pad[k * BNK:(k + 1) * BNK], table)
        emb2 = emb.reshape(BNTK // 2, 2 * D)
        out = _tc_pool_proj_ln(emb2, WaM, ba, Wp2, bp, gamma, beta,
                               out, k * (BNK // S))
    return out.reshape(B, N, DM)


# speedup vs baseline: 1.0334x; 1.0334x over previous
import functools

import jax
import jax.numpy as jnp
from jax import lax
from jax.experimental import pallas as pl
from jax.experimental.pallas import tpu as pltpu
from jax.experimental.pallas import tpu_sc as plsc

V = 100000
D = 64
DM = 512
B, N, T = 1024, 8, 64
BN = B * N
BNT = BN * T
TP = T // 2

NC, NS = 2, 16
NW = NC * NS
K = 4
BNK = BN // K
BNTK = BNT // K
PER_W = BNTK // NW
G = 512
NCH = PER_W // G
RPC = G // T
RPW = PER_W // T


def _sc_gather(ids_pad, table):
    mesh = plsc.VectorSubcoreMesh(core_axis_name="c", subcore_axis_name="s")

    @functools.partial(
        pl.kernel,
        out_type=jax.ShapeDtypeStruct((BNTK, D), jnp.float32),
        mesh=mesh,
        scratch_types=[
            pltpu.VMEM((RPW, 2 * T), jnp.int32),
            pltpu.VMEM((G,), jnp.int32),
            pltpu.VMEM((G,), jnp.int32),
            pltpu.VMEM((G, D), jnp.float32),
            pltpu.VMEM((G, D), jnp.float32),
            pltpu.SemaphoreType.DMA,
            pltpu.SemaphoreType.DMA,
        ],
        compiler_params=pltpu.CompilerParams(use_tc_tiling_on_sc=False),
    )
    def gather_kernel(ids_hbm, table_hbm, emb_hbm, idp_v, idx0, idx1,
                      rows0, rows1, sem0, sem1):
        wid = lax.axis_index("s") * NC + lax.axis_index("c")
        base = wid * PER_W
        idxs = (idx0, idx1)
        rows = (rows0, rows1)
        sems = (sem0, sem1)

        pltpu.sync_copy(ids_hbm.at[pl.ds(wid * RPW, RPW)], idp_v)

        def start_gather(c, b):
            for r in range(RPC):
                for k in range(T // 16):
                    src = idp_v[c * RPC + r, pl.ds(k * 16, 16)]
                    idxs[b][pl.ds(r * T + k * 16, 16)] = src
            pltpu.async_copy(table_hbm.at[idxs[b]], rows[b], sems[b])

        def drain_and_scatter(c, b):
            pltpu.make_async_copy(table_hbm.at[idxs[b]],
                                  rows[b], sems[b]).wait()
            pltpu.sync_copy(rows[b], emb_hbm.at[pl.ds(base + c * G, G)])

        start_gather(0, 0)

        def body(j, carry):
            c0 = 2 * j
            start_gather(c0 + 1, 1)
            drain_and_scatter(c0, 0)
            start_gather(c0 + 2, 0)
            drain_and_scatter(c0 + 1, 1)
            return carry

        lax.fori_loop(0, NCH // 2 - 1, body, 0)
        c0 = NCH - 2
        start_gather(c0 + 1, 1)
        drain_and_scatter(c0, 0)
        drain_and_scatter(c0 + 1, 1)

    return gather_kernel(ids_pad, table)


S = 512


def _tc_body(emb_ref, wam_ref, ba_ref, wp_ref, bp_ref, g_ref, b_ref, *rest):
    out_ref = rest[-1]
    e2 = emb_ref[...]
    lfull = jnp.dot(e2, wam_ref[...],
                    preferred_element_type=jnp.float32)
    lfull = jnp.clip(lfull + ba_ref[0, 0], -20.0, 20.0)
    ef = jnp.exp(lfull)
    e3 = e2.reshape(S, TP, 2 * D)
    ef3 = ef.reshape(S, TP, 2 * D)
    pooled_un = jnp.sum(e3 * ef3, axis=1)
    sef = jnp.sum(ef3, axis=1)
    z = sef[:, 0:1] + sef[:, D:D + 1]
    pooled = pooled_un / z
    out = jnp.dot(pooled, wp_ref[...],
                  preferred_element_type=jnp.float32) + bp_ref[...]
    mu = jnp.mean(out, axis=1, keepdims=True)
    var = jnp.mean((out - mu) ** 2, axis=1, keepdims=True)
    y = (out - mu) * lax.rsqrt(var + 1e-5)
    out_ref[...] = y * g_ref[...] + b_ref[...]


def _tc_pool_proj_ln(emb2, WaM, ba, Wp2, bp, gamma, beta, acc, koff):
    grid = (BNK // S,)
    in_specs = [
        pl.BlockSpec((S * TP, 2 * D), lambda i: (i, 0)),
        pl.BlockSpec((2 * D, 2 * D), lambda i: (0, 0)),
        pl.BlockSpec((1, 1), lambda i: (0, 0)),
        pl.BlockSpec((2 * D, DM), lambda i: (0, 0)),
        pl.BlockSpec((1, DM), lambda i: (0, 0)),
        pl.BlockSpec((1, DM), lambda i: (0, 0)),
        pl.BlockSpec((1, DM), lambda i: (0, 0)),
    ]
    args = [emb2, WaM, ba.reshape(1, 1), Wp2, bp.reshape(1, DM),
            gamma.reshape(1, DM), beta.reshape(1, DM)]
    aliases = {}
    if acc is not None:
        in_specs.append(pl.BlockSpec(memory_space=pl.ANY))
        args.append(acc)
        aliases = {7: 0}
    return pl.pallas_call(
        _tc_body,
        grid=grid,
        in_specs=in_specs,
        out_specs=pl.BlockSpec((S, DM), lambda i: (i + koff, 0)),
        out_shape=jax.ShapeDtypeStruct((BN, DM), jnp.float32),
        input_output_aliases=aliases,
    )(*args)


def kernel(ids, table, Wa, ba, Wp, bp, gamma, beta):
    ids_pad = jnp.pad(ids.astype(jnp.int32).reshape(BN, T),
                      ((0, 0), (0, T)))
    WaM = jnp.kron(jnp.eye(2, dtype=jnp.float32),
                   jnp.tile(Wa, (1, D)))
    Wp2 = jnp.concatenate([Wp, Wp], axis=0)
    out = None
    for k in range(K):
        emb = _sc_gather(ids_pad[k * BNK:(k + 1) * BNK], table)
        emb2 = emb.reshape(BNTK // 2, 2 * D)
        out = _tc_pool_proj_ln(emb2, WaM, ba, Wp2, bp, gamma, beta,
                               out, k * (BNK // S))
    return out.reshape(B, N, DM)
